# Initial kernel scaffold; baseline (speedup 1.0000x reference)
import functools

import jax
import jax.numpy as jnp
from jax import lax
from jax.experimental import pallas as pl
from jax.experimental.pallas import tpu as pltpu
from jax.experimental.pallas import tpu_sc as plsc

N = 10000
E = 320000
D = 128
EPS = 1e-7
MIN_NORM = 1e-15

NC = 2
NS = 16
EPC = E // NC
EPT = EPC // NS
K = 80
NCH = EPT // K
RPT = N // NS
ZR = 125


def _spmm_body(mat_hbm, src_hbm, dst_hbm, val_hbm, out_hbm,
               acc, sbuf, dbuf, vbuf, rows, zbuf, gsem):
    cid = lax.axis_index("c")
    sid = lax.axis_index("s")

    z16 = jnp.zeros((16,), jnp.float32)

    def zrow(r, carry):
        for q in range(D // 16):
            zbuf[r, pl.ds(q * 16, 16)] = z16
        return carry

    lax.fori_loop(0, ZR, zrow, 0)

    def zcopy(k, carry):
        pltpu.sync_copy(zbuf, acc.at[pl.ds(sid * RPT + k * ZR, ZR), :])
        return carry

    lax.fori_loop(0, RPT // ZR, zcopy, 0)
    plsc.subcore_barrier()

    base = cid * EPC + sid * EPT
    zi16 = jnp.zeros((16,), jnp.int32)

    def chunk(i, carry):
        off = base + i * K
        pltpu.sync_copy(src_hbm.at[pl.ds(off, K)], sbuf.at[0])
        pltpu.sync_copy(dst_hbm.at[pl.ds(off, K)], dbuf.at[0])
        pltpu.sync_copy(val_hbm.at[pl.ds(off, K)], vbuf.at[0])
        pltpu.async_copy(mat_hbm.at[sbuf.at[0]], rows.at[0], gsem).wait()

        def group(g, c2):
            r0 = g * 16
            for l in range(16):
                r = r0 + l
                sv = plsc.load_gather(vbuf, [zi16, zi16 + r])
                for q in range(D // 16):
                    sl = pl.ds(q * 16, 16)
                    rows[0, r, sl] = rows[0, r, sl] * sv
            return c2

        lax.fori_loop(0, K // 16, group, 0)
        pltpu.sync_copy(rows.at[0], acc.at[dbuf.at[0]], add=True)
        return carry

    lax.fori_loop(0, NCH, chunk, 0)
    plsc.subcore_barrier()

    def wcopy(k, carry):
        r0 = sid * RPT + k * ZR
        pltpu.sync_copy(acc.at[pl.ds(r0, ZR), :], out_hbm.at[cid, pl.ds(r0, ZR), :])
        return carry

    lax.fori_loop(0, RPT // ZR, wcopy, 0)


def _spmm_sc(mat, src, dst, val):
    mesh = plsc.VectorSubcoreMesh(
        core_axis_name="c", subcore_axis_name="s", num_cores=NC, num_subcores=NS)
    f = pl.kernel(
        _spmm_body,
        out_type=jax.ShapeDtypeStruct((NC, N, D), jnp.float32),
        mesh=mesh,
        scratch_types=[
            pltpu.VMEM_SHARED((N, D), jnp.float32),
            pltpu.VMEM((1, K), jnp.int32),
            pltpu.VMEM((1, K), jnp.int32),
            pltpu.VMEM((1, K), jnp.float32),
            pltpu.VMEM((1, K, D), jnp.float32),
            pltpu.VMEM((ZR, D), jnp.float32),
            pltpu.SemaphoreType.DMA,
        ],
    )
    return f(mat, src, dst, val)


def _row_block_call(body, *args):
    rows = 1000
    grid = (N // rows,)
    spec = pl.BlockSpec((rows, D), lambda i: (i, 0))
    out_shape = jax.ShapeDtypeStruct((N, D), jnp.float32)
    return pl.pallas_call(
        body, out_shape=out_shape, grid=grid,
        in_specs=[spec] * len(args), out_specs=spec)(*args)


def _tangent_body(x_ref, o_ref):
    xb = x_ref[...]
    col = lax.broadcasted_iota(jnp.int32, xb.shape, 1)
    xm = jnp.where(col > 0, xb, 0.0)
    s = jnp.sum(xm * xm, axis=1, keepdims=True)
    t = jnp.sqrt(1.0 + s)
    theta = jnp.maximum(t, 1.0 + EPS)
    yn = jnp.maximum(jnp.sqrt(s), MIN_NORM)
    coef = jnp.log(theta + jnp.sqrt(theta * theta - 1.0)) / yn
    o_ref[...] = xm * coef


def _add_body(a_ref, b_ref, o_ref):
    o_ref[...] = a_ref[...] + b_ref[...]


def _final_body(h_ref, a_ref, b_ref, o_ref):
    u = h_ref[...] + a_ref[...] + b_ref[...]
    col = lax.broadcasted_iota(jnp.int32, u.shape, 1)
    um = jnp.where(col > 0, u, 0.0)
    s = jnp.sum(um * um, axis=1, keepdims=True)
    xn = jnp.maximum(jnp.sqrt(s), MIN_NORM)
    e = jnp.exp(xn)
    sinh = 0.5 * (e - 1.0 / e)
    sp = (sinh / xn) * um
    s2 = jnp.sum(sp * sp, axis=1, keepdims=True)
    t2 = jnp.sqrt(jnp.maximum(1.0 + s2, EPS))
    o_ref[...] = jnp.where(col > 0, sp, t2)


def kernel(x, edge_index, adj_values):
    dst = edge_index[0]
    src = edge_index[1]
    t = _row_block_call(_tangent_body, x)
    p = _spmm_sc(t, src, dst, adj_values)
    h1 = _row_block_call(_add_body, p[0], p[1])
    q = _spmm_sc(h1, src, dst, adj_values)
    return _row_block_call(_final_body, h1, q[0], q[1])


# speedup vs baseline: 3.7940x; 3.7940x over previous
import functools

import jax
import jax.numpy as jnp
from jax import lax
from jax.experimental import pallas as pl
from jax.experimental.pallas import tpu as pltpu
from jax.experimental.pallas import tpu_sc as plsc

N = 10000
E = 320000
D = 128
EPS = 1e-7
MIN_NORM = 1e-15

NC = 2
NS = 16
EPC = E // NC
EPT = EPC // NS
K = 80
NCH = EPT // K
RPT = 624
ZR = 16


def _spmm_body(mat_hbm, src_hbm, dst_hbm, val_hbm, out_hbm,
               acc, sbuf, dbuf, vbuf, rows, zbuf, gsem):
    cid = lax.axis_index("c")
    sid = lax.axis_index("s")

    row0 = sid * RPT
    nchunks = jnp.where(sid == NS - 1, (N - (NS - 1) * RPT) // ZR, RPT // ZR)

    z16 = jnp.zeros((16,), jnp.float32)
    for r in range(ZR):
        for q in range(D // 16):
            zbuf[r, pl.ds(q * 16, 16)] = z16

    def zcopy(k, carry):
        pltpu.sync_copy(zbuf, acc.at[pl.ds(row0 + k * ZR, ZR), :])
        return carry

    lax.fori_loop(0, nchunks, zcopy, 0)
    plsc.subcore_barrier()

    base = cid * EPC + sid * EPT
    zi16 = jnp.zeros((16,), jnp.int32)

    def chunk(i, carry):
        off = base + i * K
        pltpu.sync_copy(src_hbm.at[pl.ds(off, K)], sbuf.at[0])
        pltpu.sync_copy(dst_hbm.at[pl.ds(off, K)], dbuf.at[0])
        pltpu.sync_copy(val_hbm.at[pl.ds(off, K)], vbuf.at[0])
        pltpu.async_copy(mat_hbm.at[sbuf.at[0]], rows.at[0], gsem).wait()

        def group(g, c2):
            r0 = g * 16
            vv = vbuf[0, pl.ds(r0, 16)]
            for l in range(16):
                r = r0 + l
                sv = jnp.broadcast_to(vv[l], (16,))
                for q in range(D // 16):
                    sl = pl.ds(q * 16, 16)
                    rows[0, r, sl] = rows[0, r, sl] * sv
            return c2

        lax.fori_loop(0, K // 16, group, 0)
        pltpu.sync_copy(rows.at[0], acc.at[dbuf.at[0]], add=True)
        return carry

    lax.fori_loop(0, NCH, chunk, 0)
    plsc.subcore_barrier()

    def wcopy(k, carry):
        r0 = row0 + k * ZR
        pltpu.sync_copy(acc.at[pl.ds(r0, ZR), :], out_hbm.at[cid, pl.ds(r0, ZR), :])
        return carry

    lax.fori_loop(0, nchunks, wcopy, 0)


def _spmm_sc(mat, src, dst, val):
    mesh = plsc.VectorSubcoreMesh(
        core_axis_name="c", subcore_axis_name="s", num_cores=NC, num_subcores=NS)
    f = pl.kernel(
        _spmm_body,
        out_type=jax.ShapeDtypeStruct((NC, N, D), jnp.float32),
        mesh=mesh,
        scratch_types=[
            pltpu.VMEM_SHARED((N, D), jnp.float32),
            pltpu.VMEM((1, K), jnp.int32),
            pltpu.VMEM((1, K), jnp.int32),
            pltpu.VMEM((1, K), jnp.float32),
            pltpu.VMEM((1, K, D), jnp.float32),
            pltpu.VMEM((ZR, D), jnp.float32),
            pltpu.SemaphoreType.DMA,
        ],
    )
    return f(mat, src, dst, val)


def _row_block_call(body, *args):
    rows = 1000
    grid = (N // rows,)
    spec = pl.BlockSpec((rows, D), lambda i: (i, 0))
    out_shape = jax.ShapeDtypeStruct((N, D), jnp.float32)
    return pl.pallas_call(
        body, out_shape=out_shape, grid=grid,
        in_specs=[spec] * len(args), out_specs=spec)(*args)


def _tangent_body(x_ref, o_ref):
    xb = x_ref[...]
    col = lax.broadcasted_iota(jnp.int32, xb.shape, 1)
    xm = jnp.where(col > 0, xb, 0.0)
    s = jnp.sum(xm * xm, axis=1, keepdims=True)
    t = jnp.sqrt(1.0 + s)
    theta = jnp.maximum(t, 1.0 + EPS)
    yn = jnp.maximum(jnp.sqrt(s), MIN_NORM)
    coef = jnp.log(theta + jnp.sqrt(theta * theta - 1.0)) / yn
    o_ref[...] = xm * coef


def _add_body(a_ref, b_ref, o_ref):
    o_ref[...] = a_ref[...] + b_ref[...]


def _final_body(h_ref, a_ref, b_ref, o_ref):
    u = h_ref[...] + a_ref[...] + b_ref[...]
    col = lax.broadcasted_iota(jnp.int32, u.shape, 1)
    um = jnp.where(col > 0, u, 0.0)
    s = jnp.sum(um * um, axis=1, keepdims=True)
    xn = jnp.maximum(jnp.sqrt(s), MIN_NORM)
    e = jnp.exp(xn)
    sinh = 0.5 * (e - 1.0 / e)
    sp = (sinh / xn) * um
    s2 = jnp.sum(sp * sp, axis=1, keepdims=True)
    t2 = jnp.sqrt(jnp.maximum(1.0 + s2, EPS))
    o_ref[...] = jnp.where(col > 0, sp, t2)


def kernel(x, edge_index, adj_values):
    dst = edge_index[0]
    src = edge_index[1]
    t = _row_block_call(_tangent_body, x)
    p = _spmm_sc(t, src, dst, adj_values)
    h1 = _row_block_call(_add_body, p[0], p[1])
    q = _spmm_sc(h1, src, dst, adj_values)
    return _row_block_call(_final_body, h1, q[0], q[1])
